# baseline (speedup 1.0000x reference)
import functools

import jax
import jax.numpy as jnp
from jax import lax
from jax.experimental import pallas as pl
from jax.experimental.pallas import tpu as pltpu
from jax.experimental.pallas import tpu_sc as plsc

LANES = 16
NW = 32
CN = 4


def _tc_tables(x_t, a_t, b_t, bias):
    np_, c = x_t.shape
    out = a_t.shape[1]

    def body(x_ref, a_ref, bt_ref, bias_ref, t_ref):
        xb = x_ref[...]
        t_ref[:np_, :] = (
            jnp.dot(xb, a_ref[...], preferred_element_type=jnp.float32)
            + bias_ref[...]
        )
        t_ref[np_:, :] = jnp.dot(xb, bt_ref[...], preferred_element_type=jnp.float32)

    return pl.pallas_call(
        body,
        out_shape=jax.ShapeDtypeStruct((2 * np_, out), jnp.float32),
    )(x_t, a_t, b_t, bias)


def _sc_aggregate(t, idxc, n_pad, out_dim, k):
    pw = n_pad // NW
    n_chunks = pw // CN
    ce = CN * k
    groups = out_dim // LANES
    widx = pw * k * 2

    mesh = plsc.VectorSubcoreMesh(core_axis_name="c", subcore_axis_name="s")
    D = 2

    @functools.partial(
        pl.kernel,
        mesh=mesh,
        out_type=jax.ShapeDtypeStruct((n_pad, out_dim), jnp.float32),
        scratch_types=[
            pltpu.VMEM((widx,), jnp.int32),
            [pltpu.VMEM((2 * ce, out_dim), jnp.float32)] * D,
            [pltpu.VMEM((CN, out_dim), jnp.float32)] * D,
            [pltpu.SemaphoreType.DMA] * D,
            [pltpu.SemaphoreType.DMA] * D,
        ],
    )
    def sc_kernel(t_hbm, idx_hbm, out_hbm, idx_v, g_v, o_v, sem_g, sem_o):
        wid = lax.axis_index("s") * 2 + lax.axis_index("c")
        base = wid * pw

        pltpu.sync_copy(idx_hbm.at[pl.ds(wid * widx, widx)], idx_v)

        def gather_start(ci, buf):
            pltpu.make_async_copy(
                t_hbm.at[idx_v.at[pl.ds(ci * 2 * ce, 2 * ce)]],
                g_v[buf], sem_g[buf]).start()

        def gather_wait(buf):
            pltpu.make_async_copy(
                t_hbm.at[idx_v.at[pl.ds(0, 2 * ce)]],
                g_v[buf], sem_g[buf]).wait()

        gather_start(0, 0)

        def iteration(ci, b):
            @pl.when(ci + 1 < n_chunks)
            def _():
                gather_start(ci + 1, 1 - b)

            gather_wait(b)

            @pl.when(ci >= D)
            def _():
                pltpu.make_async_copy(
                    o_v[b], out_hbm.at[pl.ds(base, CN)], sem_o[b]).wait()

            na = 4
            for n in range(CN):
                for g in range(groups):
                    sl = pl.ds(g * LANES, LANES)
                    acc = [g_v[b][n * k + kk, sl] + g_v[b][ce + n * k + kk, sl]
                           for kk in range(na)]
                    for kk in range(na, k):
                        acc[kk % na] = jnp.maximum(
                            acc[kk % na],
                            g_v[b][n * k + kk, sl] + g_v[b][ce + n * k + kk, sl])
                    m = jnp.maximum(jnp.maximum(acc[0], acc[1]),
                                    jnp.maximum(acc[2], acc[3]))
                    o_v[b][n, sl] = jnp.maximum(m, 0.0)

            ns = base + ci * CN
            pltpu.make_async_copy(o_v[b], out_hbm.at[pl.ds(ns, CN)], sem_o[b]).start()

        def body(p, carry):
            for j in range(D):
                iteration(p * D + j, j)
            return carry

        lax.fori_loop(0, n_chunks // D, body, 0)
        for ci in range((n_chunks // D) * D, n_chunks):
            iteration(ci, ci % D)

        for d in range(D):
            pltpu.make_async_copy(o_v[d], out_hbm.at[pl.ds(base, CN)], sem_o[d]).wait()

    return sc_kernel(t, idxc)


def kernel(x, edge_index, W, b):
    bb, c, n, _ = x.shape
    k = edge_index.shape[3]
    out_dim = W.shape[0]

    n_pad = ((n + NW * CN - 1) // (NW * CN)) * (NW * CN)

    x_t = jnp.transpose(x.reshape(c, n))
    x_t = jnp.pad(x_t, ((0, n_pad - n), (0, 0)))

    w1 = W[:, :c]
    w2 = W[:, c:]
    a_t = jnp.transpose(w1 - w2)
    b_t = jnp.transpose(w2)
    bias = b.reshape(1, out_dim)

    t = _tc_tables(x_t, a_t, b_t, bias)

    ei = edge_index.reshape(2, n * k)
    pad_e = n_pad * k - n * k
    ce = CN * k
    idx_i = jnp.pad(ei[1], (0, pad_e)).reshape(-1, ce)
    idx_j = jnp.pad(ei[0], (0, pad_e)).reshape(-1, ce) + n_pad
    idxc = jnp.concatenate([idx_i, idx_j], axis=1).reshape(-1)

    out_full = _sc_aggregate(t, idxc, n_pad, out_dim, k)

    out = jnp.transpose(out_full[:n, :])
    return out.reshape(bb, out_dim, n, 1)
